# baseline (speedup 1.0000x reference)
import functools

import jax
import jax.numpy as jnp
from jax.experimental import pallas as pl
from jax.experimental.pallas import tpu as pltpu

B = 8
N = 4096
S = 512
K = 32
D = 64
MLP_CH = [64, 64, 128]
EPS = 1e-5


def _fps_body(x_ref, y_ref, z_ref, nx_ref, ny_ref, nz_ref, dist_ref):
    x = x_ref[...]
    y = y_ref[...]
    z = z_ref[...]
    iota = jax.lax.broadcasted_iota(jnp.int32, (B, N), 1)
    lane = jax.lax.broadcasted_iota(jnp.int32, (B, 128), 1)
    dist_ref[...] = jnp.full((B, N), 1e10, jnp.float32)

    def body(i, state):
        far, bx, by, bz = state
        onehot = iota == far
        cx = jnp.max(jnp.where(onehot, x, -jnp.inf), axis=1, keepdims=True)
        cy = jnp.max(jnp.where(onehot, y, -jnp.inf), axis=1, keepdims=True)
        cz = jnp.max(jnp.where(onehot, z, -jnp.inf), axis=1, keepdims=True)
        sel = lane == i
        bx = jnp.where(sel, cx, bx)
        by = jnp.where(sel, cy, by)
        bz = jnp.where(sel, cz, bz)
        dx = x - cx
        dy = y - cy
        dz = z - cz
        d = dx * dx + dy * dy + dz * dz
        dmin = jnp.minimum(dist_ref[...], d)
        dist_ref[...] = dmin
        m = jnp.max(dmin, axis=1, keepdims=True)
        far_new = jnp.min(jnp.where(dmin == m, iota, N), axis=1, keepdims=True)
        return far_new, bx, by, bz

    far = jnp.zeros((B, 1), jnp.int32)
    zbuf = jnp.zeros((B, 128), jnp.float32)
    for j in range(S // 128):
        far, bx, by, bz = jax.lax.fori_loop(0, 128, body, (far, zbuf, zbuf, zbuf))
        nx_ref[:, j * 128:(j + 1) * 128] = bx
        ny_ref[:, j * 128:(j + 1) * 128] = by
        nz_ref[:, j * 128:(j + 1) * 128] = bz


def _fps(x, y, z):
    out = pl.pallas_call(
        _fps_body,
        out_shape=[jax.ShapeDtypeStruct((B, S), jnp.float32)] * 3,
        scratch_shapes=[pltpu.VMEM((B, N), jnp.float32)],
    )(x, y, z)
    return out


SBLK = 128


def _knn_body(x_ref, y_ref, z_ref, cx_ref, cy_ref, cz_ref, di_ref, thr_ref):
    x = x_ref[0]
    cx = cx_ref[0]
    dx = x - cx
    dy = y_ref[0] - cy_ref[0]
    dz = z_ref[0] - cz_ref[0]
    d = dx * dx + dy * dy + dz * dz
    di = jax.lax.bitcast_convert_type(d, jnp.int32)
    di_ref[...] = di[None]
    acc = jnp.zeros((SBLK, 1), jnp.int32)
    for b in range(30, -1, -1):
        trial = acc | (1 << b)
        cnt = jnp.sum((di < trial).astype(jnp.int32), axis=1, keepdims=True)
        acc = jnp.where(cnt < K, trial, acc)
    thr_ref[...] = acc[None]


def _knn_thresh(x, y, z, cxg, cyg, czg):
    nsb = S // SBLK
    grid = (B, nsb)
    return pl.pallas_call(
        _knn_body,
        grid=grid,
        in_specs=[
            pl.BlockSpec((1, 1, N), lambda b, s: (b, 0, 0)),
            pl.BlockSpec((1, 1, N), lambda b, s: (b, 0, 0)),
            pl.BlockSpec((1, 1, N), lambda b, s: (b, 0, 0)),
            pl.BlockSpec((1, SBLK, 1), lambda b, s: (b * nsb + s, 0, 0)),
            pl.BlockSpec((1, SBLK, 1), lambda b, s: (b * nsb + s, 0, 0)),
            pl.BlockSpec((1, SBLK, 1), lambda b, s: (b * nsb + s, 0, 0)),
        ],
        out_specs=[
            pl.BlockSpec((1, SBLK, N), lambda b, s: (b, s, 0)),
            pl.BlockSpec((1, SBLK, 1), lambda b, s: (b * nsb + s, 0, 0)),
        ],
        out_shape=[
            jax.ShapeDtypeStruct((B, S, N), jnp.int32),
            jax.ShapeDtypeStruct((B * nsb, SBLK, 1), jnp.int32),
        ],
    )(x, y, z, cxg, cyg, czg)


def kernel(xyz, points, params):
    x = xyz[:, :, 0]
    y = xyz[:, :, 1]
    z = xyz[:, :, 2]
    nx, ny, nz = _fps(x, y, z)
    new_xyz = jnp.stack([nx, ny, nz], axis=-1)

    def _cg(a):
        return a.reshape(B * (S // SBLK), SBLK, 1)

    di, thrg = _knn_thresh(x[:, None, :], y[:, None, :], z[:, None, :],
                           _cg(nx), _cg(ny), _cg(nz))
    thr = thrg.reshape(B, S)

    lt = di < thr[..., None]
    eq = di == thr[..., None]
    need = K - jnp.sum(lt, axis=-1, keepdims=True)
    takeeq = eq & (jnp.cumsum(eq.astype(jnp.int32), axis=-1) <= need)
    mask = lt | takeeq
    idx = jnp.argsort(jnp.where(mask, 0, 1), axis=-1, stable=True)[..., :K]
    idx_flat = idx.reshape(B, -1)
    gxyz = jnp.take_along_axis(
        xyz, jnp.broadcast_to(idx_flat[..., None], (B, S * K, 3)), axis=1
    ).reshape(B, S, K, 3)
    gxyz = gxyz - new_xyz[:, :, None, :]
    gpts = jnp.take_along_axis(
        points, jnp.broadcast_to(idx_flat[..., None], (B, S * K, D)), axis=1
    ).reshape(B, S, K, D)
    feat = jnp.concatenate([gxyz, gpts], axis=-1)
    xt = jnp.transpose(feat, (0, 3, 2, 1))
    for i in range(len(MLP_CH)):
        xt = (
            jnp.einsum("oc,bcks->boks", params[f"w{i}"], xt)
            + params[f"b{i}"][None, :, None, None]
        )
        mean = jnp.mean(xt, axis=(0, 2, 3), keepdims=True)
        var = jnp.var(xt, axis=(0, 2, 3), keepdims=True)
        xt = (xt - mean) / jnp.sqrt(var + EPS)
        xt = xt * params[f"g{i}"][None, :, None, None] + params[f"be{i}"][None, :, None, None]
        xt = jax.nn.relu(xt)
    new_points_out = jnp.transpose(jnp.max(xt, axis=2), (0, 2, 1))
    return (new_xyz, new_points_out)


# speedup vs baseline: 89.9217x; 89.9217x over previous
import functools

import jax
import jax.numpy as jnp
from jax import lax
from jax.experimental import pallas as pl
from jax.experimental.pallas import tpu as pltpu
from jax.experimental.pallas import tpu_sc as plsc

B = 8
N = 4096
S = 512
K = 32
D = 64
MLP_CH = [64, 64, 128]
EPS = 1e-5


def _fps_body(x_ref, y_ref, z_ref, nx_ref, ny_ref, nz_ref, dist_ref):
    x = x_ref[...]
    y = y_ref[...]
    z = z_ref[...]
    iota = jax.lax.broadcasted_iota(jnp.int32, (B, N), 1)
    lane = jax.lax.broadcasted_iota(jnp.int32, (B, 128), 1)
    dist_ref[...] = jnp.full((B, N), 1e10, jnp.float32)

    def body(i, state):
        far, bx, by, bz = state
        onehot = iota == far
        cx = jnp.max(jnp.where(onehot, x, -jnp.inf), axis=1, keepdims=True)
        cy = jnp.max(jnp.where(onehot, y, -jnp.inf), axis=1, keepdims=True)
        cz = jnp.max(jnp.where(onehot, z, -jnp.inf), axis=1, keepdims=True)
        sel = lane == i
        bx = jnp.where(sel, cx, bx)
        by = jnp.where(sel, cy, by)
        bz = jnp.where(sel, cz, bz)
        dx = x - cx
        dy = y - cy
        dz = z - cz
        d = dx * dx + dy * dy + dz * dz
        dmin = jnp.minimum(dist_ref[...], d)
        dist_ref[...] = dmin
        m = jnp.max(dmin, axis=1, keepdims=True)
        far_new = jnp.min(jnp.where(dmin == m, iota, N), axis=1, keepdims=True)
        return far_new, bx, by, bz

    far = jnp.zeros((B, 1), jnp.int32)
    zbuf = jnp.zeros((B, 128), jnp.float32)
    for j in range(S // 128):
        far, bx, by, bz = jax.lax.fori_loop(0, 128, body, (far, zbuf, zbuf, zbuf))
        nx_ref[:, j * 128:(j + 1) * 128] = bx
        ny_ref[:, j * 128:(j + 1) * 128] = by
        nz_ref[:, j * 128:(j + 1) * 128] = bz


def _fps(x, y, z):
    out = pl.pallas_call(
        _fps_body,
        out_shape=[jax.ShapeDtypeStruct((B, S), jnp.float32)] * 3,
        scratch_shapes=[pltpu.VMEM((B, N), jnp.float32)],
    )(x, y, z)
    return out


SBLK = 128


def _knn_body(x_ref, y_ref, z_ref, cx_ref, cy_ref, cz_ref, di_ref, thr_ref):
    x = x_ref[0]
    cx = cx_ref[0]
    dx = x - cx
    dy = y_ref[0] - cy_ref[0]
    dz = z_ref[0] - cz_ref[0]
    d = dx * dx + dy * dy + dz * dz
    di = jax.lax.bitcast_convert_type(d, jnp.int32)
    di_ref[...] = di[None]
    acc = jnp.zeros((SBLK, 1), jnp.int32)
    for b in range(30, -1, -1):
        trial = acc | (1 << b)
        cnt = jnp.sum((di < trial).astype(jnp.int32), axis=1, keepdims=True)
        acc = jnp.where(cnt < K, trial, acc)
    thr_ref[...] = acc[None]


def _knn_thresh(x, y, z, cxg, cyg, czg):
    nsb = S // SBLK
    grid = (B, nsb)
    return pl.pallas_call(
        _knn_body,
        grid=grid,
        in_specs=[
            pl.BlockSpec((1, 1, N), lambda b, s: (b, 0, 0)),
            pl.BlockSpec((1, 1, N), lambda b, s: (b, 0, 0)),
            pl.BlockSpec((1, 1, N), lambda b, s: (b, 0, 0)),
            pl.BlockSpec((1, SBLK, 1), lambda b, s: (b * nsb + s, 0, 0)),
            pl.BlockSpec((1, SBLK, 1), lambda b, s: (b * nsb + s, 0, 0)),
            pl.BlockSpec((1, SBLK, 1), lambda b, s: (b * nsb + s, 0, 0)),
        ],
        out_specs=[
            pl.BlockSpec((1, SBLK, N), lambda b, s: (b, s, 0)),
            pl.BlockSpec((1, SBLK, 1), lambda b, s: (b * nsb + s, 0, 0)),
        ],
        out_shape=[
            jax.ShapeDtypeStruct((B, S, N), jnp.int32),
            jax.ShapeDtypeStruct((B * nsb, SBLK, 1), jnp.int32),
        ],
    )(x, y, z, cxg, cyg, czg)


XP = 16


def _sc_group_gather(di_f, thr, pts, xyzp, cen):
    info = plsc.get_sparse_core_info()
    NC, NS = info.num_cores, info.num_subcores
    NW = NC * NS
    rpw = (B * S) // NW
    mesh = plsc.VectorSubcoreMesh(core_axis_name="c", subcore_axis_name="s")

    @functools.partial(
        pl.kernel, mesh=mesh,
        compiler_params=pltpu.CompilerParams(
            needs_layout_passes=False, use_tc_tiling_on_sc=False),
        out_type=[jax.ShapeDtypeStruct((B * S, K, D), jnp.float32),
                  jax.ShapeDtypeStruct((B * S, K, XP), jnp.float32)],
        scratch_types=[
            pltpu.VMEM((N,), jnp.int32),
            pltpu.VMEM((rpw,), jnp.int32),
            pltpu.VMEM((K,), jnp.int32),
            pltpu.VMEM((K, D), jnp.float32),
            pltpu.VMEM((K, XP), jnp.float32),
            pltpu.VMEM((XP,), jnp.float32),
            pltpu.SemaphoreType.DMA,
        ],
    )
    def k(di_hbm, thr_hbm, pts_hbm, xyzp_hbm, cen_hbm, gp_hbm, gx_hbm,
          dirow, thrv_ref, selg, prow, xrow, cenv, sem):
        wid = lax.axis_index("s") * NC + lax.axis_index("c")
        base = wid * rpw
        pltpu.sync_copy(thr_hbm.at[pl.ds(base, rpw)], thrv_ref)
        iota16 = lax.iota(jnp.int32, 16)

        def row_body(r, carry):
            g = base + r
            b = g // S
            bn = b * N
            pltpu.sync_copy(di_hbm.at[g], dirow)
            thrv = plsc.load_gather(thrv_ref, [jnp.full((16,), r, jnp.int32)])

            def chunk_lt(c, off):
                v = dirow[pl.ds(c * 16, 16)]
                m = v < thrv
                rank = off + plsc.cumsum(m.astype(jnp.int32)) - 1
                plsc.store_scatter(selg, [rank], c * 16 + iota16 + bn, mask=m)
                return off + plsc.all_reduce_population_count(m)

            off = lax.fori_loop(0, N // 16, chunk_lt,
                                jnp.zeros((16,), jnp.int32))

            def chunk_eq(c, off):
                v = dirow[pl.ds(c * 16, 16)]
                m = v == thrv
                rank = off + plsc.cumsum(m.astype(jnp.int32)) - 1
                m2 = m & (rank < K)
                plsc.store_scatter(selg, [rank], c * 16 + iota16 + bn, mask=m2)
                return off + plsc.all_reduce_population_count(m2)

            lax.fori_loop(0, N // 16, chunk_eq, off)

            pltpu.async_copy(pts_hbm.at[selg], prow, sem).wait()
            pltpu.sync_copy(prow, gp_hbm.at[g])
            pltpu.async_copy(xyzp_hbm.at[selg], xrow, sem).wait()
            pltpu.sync_copy(cen_hbm.at[g], cenv)
            cv = cenv[...]
            for j in range(K):
                xrow[j] = xrow[j] - cv
            pltpu.sync_copy(xrow, gx_hbm.at[g])
            return carry

        lax.fori_loop(0, rpw, row_body, 0)

    return k(di_f, thr, pts, xyzp, cen)


def kernel(xyz, points, params):
    x = xyz[:, :, 0]
    y = xyz[:, :, 1]
    z = xyz[:, :, 2]
    nx, ny, nz = _fps(x, y, z)
    new_xyz = jnp.stack([nx, ny, nz], axis=-1)

    def _cg(a):
        return a.reshape(B * (S // SBLK), SBLK, 1)

    di, thrg = _knn_thresh(x[:, None, :], y[:, None, :], z[:, None, :],
                           _cg(nx), _cg(ny), _cg(nz))
    thr = thrg.reshape(B, S)

    xyzp = jnp.pad(xyz, ((0, 0), (0, 0), (0, XP - 3))).reshape(B * N, XP)
    cen = jnp.pad(new_xyz, ((0, 0), (0, 0), (0, XP - 3))).reshape(B * S, XP)
    gp, gx = _sc_group_gather(
        di.reshape(B * S, N), thr.reshape(B * S), points.reshape(B * N, D),
        xyzp, cen)
    gpts = gp.reshape(B, S, K, D)
    gxyz = gx.reshape(B, S, K, XP)[..., :3]

    feat = jnp.concatenate([gxyz, gpts], axis=-1)
    xt = jnp.transpose(feat, (0, 3, 2, 1))
    for i in range(len(MLP_CH)):
        xt = (
            jnp.einsum("oc,bcks->boks", params[f"w{i}"], xt)
            + params[f"b{i}"][None, :, None, None]
        )
        mean = jnp.mean(xt, axis=(0, 2, 3), keepdims=True)
        var = jnp.var(xt, axis=(0, 2, 3), keepdims=True)
        xt = (xt - mean) / jnp.sqrt(var + EPS)
        xt = xt * params[f"g{i}"][None, :, None, None] + params[f"be{i}"][None, :, None, None]
        xt = jax.nn.relu(xt)
    new_points_out = jnp.transpose(jnp.max(xt, axis=2), (0, 2, 1))
    return (new_xyz, new_points_out)
